# baseline (speedup 1.0000x reference)
import jax
import jax.numpy as jnp
from jax import lax
from jax.experimental import pallas as pl
from jax.experimental.pallas import tpu as pltpu
from jax.experimental.pallas import tpu_sc as plsc

T_MAX = 1000
COND_DIM = 128
BATCH = 16384

NC = 2
NS = 16
LANES = 16
NW = NC * NS
B_PER_W = BATCH // NW
CHUNK = 128
N_CHUNKS = B_PER_W // CHUNK


def _emb_lookup_body(
    t_hbm, table_hbm, out_hbm, idx_flat, idx2, rows, gs0, gs1, gs2, gs3, wsem
):
    gsems = [gs0, gs1, gs2, gs3]
    wid = lax.axis_index("s") * NC + lax.axis_index("c")
    base = wid * B_PER_W

    pltpu.sync_copy(t_hbm.at[pl.ds(base, B_PER_W)], idx_flat)

    gathers = []
    for j in range(N_CHUNKS):
        for i in range(CHUNK // LANES):
            v = idx_flat[pl.ds(j * CHUNK + i * LANES, LANES)]
            v = jnp.where(v == 0, T_MAX - 1, v - 1)
            idx2[j, pl.ds(i * LANES, LANES)] = v
        gathers.append(
            pltpu.async_copy(
                table_hbm.at[idx2.at[j]], rows.at[pl.ds(j * CHUNK, CHUNK)], gsems[j]
            )
        )

    writes = []
    for j in range(N_CHUNKS):
        gathers[j].wait()
        writes.append(
            pltpu.async_copy(
                rows.at[pl.ds(j * CHUNK, CHUNK)],
                out_hbm.at[pl.ds(base + j * CHUNK, CHUNK)],
                wsem,
            )
        )
    for w in writes:
        w.wait()


@jax.jit
def kernel(t, time_emb):
    mesh = plsc.VectorSubcoreMesh(
        core_axis_name="c", subcore_axis_name="s", num_cores=NC, num_subcores=NS
    )
    run = pl.kernel(
        _emb_lookup_body,
        out_type=jax.ShapeDtypeStruct((BATCH, COND_DIM), jnp.float32),
        mesh=mesh,
        scratch_types=[
            pltpu.VMEM((B_PER_W,), jnp.int32),
            pltpu.VMEM((N_CHUNKS, CHUNK), jnp.int32),
            pltpu.VMEM((B_PER_W, COND_DIM), jnp.float32),
            pltpu.SemaphoreType.DMA,
            pltpu.SemaphoreType.DMA,
            pltpu.SemaphoreType.DMA,
            pltpu.SemaphoreType.DMA,
            pltpu.SemaphoreType.DMA,
        ],
    )
    return run(t, time_emb)


# speedup vs baseline: 1.0727x; 1.0727x over previous
import jax
import jax.numpy as jnp
from jax import lax
from jax.experimental import pallas as pl
from jax.experimental.pallas import tpu as pltpu
from jax.experimental.pallas import tpu_sc as plsc

T_MAX = 1000
COND_DIM = 128
BATCH = 16384

NC = 2
NS = 16
LANES = 16
NW = NC * NS
B_PER_W = BATCH // NW
CHUNK = 128
N_CHUNKS = B_PER_W // CHUNK


def _emb_lookup_body(t_hbm, table_hbm, out_hbm, idx_flat, idx2, rows, sem):
    wid = lax.axis_index("s") * NC + lax.axis_index("c")
    base = wid * B_PER_W

    pltpu.sync_copy(t_hbm.at[pl.ds(base, B_PER_W)], idx_flat)

    def adjust(i, _):
        v = idx_flat[pl.ds(i * LANES, LANES)]
        v = jnp.where(v == 0, T_MAX - 1, v - 1)
        idx2[i // (CHUNK // LANES), pl.ds((i % (CHUNK // LANES)) * LANES, LANES)] = v
        return _

    lax.fori_loop(0, B_PER_W // LANES, adjust, 0, unroll=False)

    copies = []
    for j in range(N_CHUNKS):
        copies.append(
            pltpu.async_copy(
                table_hbm.at[idx2.at[j]], rows.at[pl.ds(j * CHUNK, CHUNK)], sem
            )
        )
    for c in copies:
        c.wait()

    pltpu.sync_copy(rows, out_hbm.at[pl.ds(base, B_PER_W)])


@jax.jit
def kernel(t, time_emb):
    mesh = plsc.VectorSubcoreMesh(
        core_axis_name="c", subcore_axis_name="s", num_cores=NC, num_subcores=NS
    )
    run = pl.kernel(
        _emb_lookup_body,
        out_type=jax.ShapeDtypeStruct((BATCH, COND_DIM), jnp.float32),
        mesh=mesh,
        scratch_types=[
            pltpu.VMEM((B_PER_W,), jnp.int32),
            pltpu.VMEM((N_CHUNKS, CHUNK), jnp.int32),
            pltpu.VMEM((B_PER_W, COND_DIM), jnp.float32),
            pltpu.SemaphoreType.DMA,
        ],
    )
    return run(t, time_emb)


# speedup vs baseline: 1.1139x; 1.0384x over previous
import jax
import jax.numpy as jnp
from jax import lax
from jax.experimental import pallas as pl
from jax.experimental.pallas import tpu as pltpu
from jax.experimental.pallas import tpu_sc as plsc

T_MAX = 1000
COND_DIM = 128
BATCH = 16384

NC = 2
NS = 16
LANES = 16
NW = NC * NS
B_PER_W = BATCH // NW
CHUNK = 128
N_CHUNKS = B_PER_W // CHUNK


T_PAD = 1024
STAGE_ROWS = T_PAD // NS


def _emb_lookup_body(t_hbm, table_hbm, out_hbm, idx_flat, idx2, rows, tbl_sp, sem):
    c = lax.axis_index("c")
    s = lax.axis_index("s")
    wid = s * NC + c
    base = wid * B_PER_W

    pltpu.sync_copy(
        table_hbm.at[pl.ds(s * STAGE_ROWS, STAGE_ROWS)],
        rows.at[pl.ds(0, STAGE_ROWS)],
    )
    pltpu.sync_copy(
        rows.at[pl.ds(0, STAGE_ROWS)],
        tbl_sp.at[pl.ds(s * STAGE_ROWS, STAGE_ROWS)],
    )

    pltpu.sync_copy(t_hbm.at[pl.ds(base, B_PER_W)], idx_flat)

    def adjust(i, _):
        v = idx_flat[pl.ds(i * LANES, LANES)]
        v = jnp.where(v == 0, T_MAX - 1, v - 1)
        idx2[i // (CHUNK // LANES), pl.ds((i % (CHUNK // LANES)) * LANES, LANES)] = v
        return _

    lax.fori_loop(0, B_PER_W // LANES, adjust, 0, unroll=False)

    plsc.subcore_barrier()

    copies = []
    for j in range(N_CHUNKS):
        copies.append(
            pltpu.async_copy(
                tbl_sp.at[idx2.at[j]], rows.at[pl.ds(j * CHUNK, CHUNK)], sem
            )
        )
    for c in copies:
        c.wait()

    pltpu.sync_copy(rows, out_hbm.at[pl.ds(base, B_PER_W)])


@jax.jit
def kernel(t, time_emb):
    table = jnp.pad(time_emb, ((0, T_PAD - T_MAX), (0, 0)))
    mesh = plsc.VectorSubcoreMesh(
        core_axis_name="c", subcore_axis_name="s", num_cores=NC, num_subcores=NS
    )
    run = pl.kernel(
        _emb_lookup_body,
        out_type=jax.ShapeDtypeStruct((BATCH, COND_DIM), jnp.float32),
        mesh=mesh,
        scratch_types=[
            pltpu.VMEM((B_PER_W,), jnp.int32),
            pltpu.VMEM((N_CHUNKS, CHUNK), jnp.int32),
            pltpu.VMEM((B_PER_W, COND_DIM), jnp.float32),
            pltpu.VMEM_SHARED((T_PAD, COND_DIM), jnp.float32),
            pltpu.SemaphoreType.DMA,
        ],
    )
    return run(t, table)


# speedup vs baseline: 1.1562x; 1.0380x over previous
import jax
import jax.numpy as jnp
from jax import lax
from jax.experimental import pallas as pl
from jax.experimental.pallas import tpu as pltpu
from jax.experimental.pallas import tpu_sc as plsc

T_MAX = 1000
COND_DIM = 128
BATCH = 16384

NC = 2
NS = 16
LANES = 16
NW = NC * NS
B_PER_W = BATCH // NW
CHUNK = 128
N_CHUNKS = B_PER_W // CHUNK

STAGE_ROWS = 64
LAST_ROWS = T_MAX - (NS - 1) * STAGE_ROWS


def _emb_lookup_body(t_hbm, table_hbm, out_hbm, idx_flat, idx2, rows, tbl_sp, wsem):
    core = lax.axis_index("c")
    s = lax.axis_index("s")
    wid = s * NC + core
    base = wid * B_PER_W

    @pl.when(s < NS - 1)
    def _stage_main():
        pltpu.sync_copy(
            table_hbm.at[pl.ds(s * STAGE_ROWS, STAGE_ROWS)],
            rows.at[pl.ds(0, STAGE_ROWS)],
        )
        pltpu.sync_copy(
            rows.at[pl.ds(0, STAGE_ROWS)],
            tbl_sp.at[pl.ds(s * STAGE_ROWS, STAGE_ROWS)],
        )

    @pl.when(s == NS - 1)
    def _stage_tail():
        pltpu.sync_copy(
            table_hbm.at[pl.ds((NS - 1) * STAGE_ROWS, LAST_ROWS)],
            rows.at[pl.ds(0, LAST_ROWS)],
        )
        pltpu.sync_copy(
            rows.at[pl.ds(0, LAST_ROWS)],
            tbl_sp.at[pl.ds((NS - 1) * STAGE_ROWS, LAST_ROWS)],
        )

    pltpu.sync_copy(t_hbm.at[pl.ds(base, B_PER_W)], idx_flat)

    def adjust(i, _):
        v = idx_flat[pl.ds(i * LANES, LANES)]
        v = jnp.where(v == 0, T_MAX - 1, v - 1)
        idx2[i // (CHUNK // LANES), pl.ds((i % (CHUNK // LANES)) * LANES, LANES)] = v
        return _

    lax.fori_loop(0, B_PER_W // LANES, adjust, 0, unroll=False)

    plsc.subcore_barrier()

    def chunk_step(j, _):
        pltpu.sync_copy(tbl_sp.at[idx2.at[j]], rows.at[pl.ds(j * CHUNK, CHUNK)])
        pltpu.async_copy(
            rows.at[pl.ds(j * CHUNK, CHUNK)],
            out_hbm.at[pl.ds(base + j * CHUNK, CHUNK)],
            wsem,
        )
        return _

    lax.fori_loop(0, N_CHUNKS, chunk_step, 0, unroll=False)

    def drain(j, _):
        pltpu.make_async_copy(
            out_hbm.at[pl.ds(0, CHUNK)], rows.at[pl.ds(0, CHUNK)], wsem
        ).wait()
        return _

    lax.fori_loop(0, N_CHUNKS, drain, 0, unroll=False)


@jax.jit
def kernel(t, time_emb):
    mesh = plsc.VectorSubcoreMesh(
        core_axis_name="c", subcore_axis_name="s", num_cores=NC, num_subcores=NS
    )
    run = pl.kernel(
        _emb_lookup_body,
        out_type=jax.ShapeDtypeStruct((BATCH, COND_DIM), jnp.float32),
        mesh=mesh,
        scratch_types=[
            pltpu.VMEM((B_PER_W,), jnp.int32),
            pltpu.VMEM((N_CHUNKS, CHUNK), jnp.int32),
            pltpu.VMEM((B_PER_W, COND_DIM), jnp.float32),
            pltpu.VMEM_SHARED((T_MAX, COND_DIM), jnp.float32),
            pltpu.SemaphoreType.DMA,
        ],
    )
    return run(t, time_emb)


# speedup vs baseline: 1.1881x; 1.0277x over previous
import jax
import jax.numpy as jnp
from jax import lax
from jax.experimental import pallas as pl
from jax.experimental.pallas import tpu as pltpu
from jax.experimental.pallas import tpu_sc as plsc

T_MAX = 1000
COND_DIM = 128
BATCH = 16384

NC = 2
NS = 16
LANES = 16
NW = NC * NS
B_PER_W = BATCH // NW
CHUNK = 128
N_CHUNKS = B_PER_W // CHUNK

STAGE_ROWS = 64
LAST_ROWS = T_MAX - (NS - 1) * STAGE_ROWS


def _emb_lookup_body(t_hbm, table_hbm, out_hbm, idx_flat, idx2, rows, tbl_sp, wsem):
    core = lax.axis_index("c")
    s = lax.axis_index("s")
    wid = s * NC + core
    base = wid * B_PER_W

    row0 = jnp.where(s == NS - 1, T_MAX - STAGE_ROWS, s * STAGE_ROWS)
    stage1 = pltpu.async_copy(
        table_hbm.at[pl.ds(row0, STAGE_ROWS)], rows.at[pl.ds(0, STAGE_ROWS)], wsem
    )

    pltpu.sync_copy(t_hbm.at[pl.ds(base, B_PER_W)], idx_flat)
    stage1.wait()
    stage2 = pltpu.async_copy(
        rows.at[pl.ds(0, STAGE_ROWS)], tbl_sp.at[pl.ds(row0, STAGE_ROWS)], wsem
    )

    def adjust(i, _):
        v = idx_flat[pl.ds(i * LANES, LANES)]
        v = jnp.where(v == 0, T_MAX - 1, v - 1)
        idx2[i // (CHUNK // LANES), pl.ds((i % (CHUNK // LANES)) * LANES, LANES)] = v
        return _

    lax.fori_loop(0, B_PER_W // LANES, adjust, 0, unroll=False)
    stage2.wait()

    plsc.subcore_barrier()

    def chunk_step(j, _):
        pltpu.sync_copy(tbl_sp.at[idx2.at[j]], rows.at[pl.ds(j * CHUNK, CHUNK)])
        pltpu.async_copy(
            rows.at[pl.ds(j * CHUNK, CHUNK)],
            out_hbm.at[pl.ds(base + j * CHUNK, CHUNK)],
            wsem,
        )
        return _

    lax.fori_loop(0, N_CHUNKS, chunk_step, 0, unroll=False)

    def drain(j, _):
        pltpu.make_async_copy(
            out_hbm.at[pl.ds(0, CHUNK)], rows.at[pl.ds(0, CHUNK)], wsem
        ).wait()
        return _

    lax.fori_loop(0, N_CHUNKS, drain, 0, unroll=False)


@jax.jit
def kernel(t, time_emb):
    mesh = plsc.VectorSubcoreMesh(
        core_axis_name="c", subcore_axis_name="s", num_cores=NC, num_subcores=NS
    )
    run = pl.kernel(
        _emb_lookup_body,
        out_type=jax.ShapeDtypeStruct((BATCH, COND_DIM), jnp.float32),
        mesh=mesh,
        scratch_types=[
            pltpu.VMEM((B_PER_W,), jnp.int32),
            pltpu.VMEM((N_CHUNKS, CHUNK), jnp.int32),
            pltpu.VMEM((B_PER_W, COND_DIM), jnp.float32),
            pltpu.VMEM_SHARED((T_MAX, COND_DIM), jnp.float32),
            pltpu.SemaphoreType.DMA,
        ],
    )
    return run(t, time_emb)


# speedup vs baseline: 1.1923x; 1.0035x over previous
import jax
import jax.numpy as jnp
from jax import lax
from jax.experimental import pallas as pl
from jax.experimental.pallas import tpu as pltpu
from jax.experimental.pallas import tpu_sc as plsc

T_MAX = 1000
COND_DIM = 128
BATCH = 16384

NC = 2
NS = 16
LANES = 16
NW = NC * NS
B_PER_W = BATCH // NW
CHUNK = 128
N_CHUNKS = B_PER_W // CHUNK

STAGE_ROWS = 64
LAST_ROWS = T_MAX - (NS - 1) * STAGE_ROWS


def _emb_lookup_body(
    t_hbm, table_hbm, out_hbm, idx_flat, idx2, rows, tbl_sp, gs0, gs1, gs2, gs3, wsem
):
    gsems = [gs0, gs1, gs2, gs3]
    core = lax.axis_index("c")
    s = lax.axis_index("s")
    wid = s * NC + core
    base = wid * B_PER_W

    row0 = jnp.where(s == NS - 1, T_MAX - STAGE_ROWS, s * STAGE_ROWS)
    stage1 = pltpu.async_copy(
        table_hbm.at[pl.ds(row0, STAGE_ROWS)], rows.at[pl.ds(0, STAGE_ROWS)], wsem
    )

    pltpu.sync_copy(t_hbm.at[pl.ds(base, B_PER_W)], idx_flat)
    stage1.wait()
    stage2 = pltpu.async_copy(
        rows.at[pl.ds(0, STAGE_ROWS)], tbl_sp.at[pl.ds(row0, STAGE_ROWS)], wsem
    )

    def adjust(i, _):
        v = idx_flat[pl.ds(i * LANES, LANES)]
        v = jnp.where(v == 0, T_MAX - 1, v - 1)
        idx2[i // (CHUNK // LANES), pl.ds((i % (CHUNK // LANES)) * LANES, LANES)] = v
        return _

    lax.fori_loop(0, B_PER_W // LANES, adjust, 0, unroll=False)
    stage2.wait()

    plsc.subcore_barrier()

    gathers = [
        pltpu.async_copy(
            tbl_sp.at[idx2.at[j]], rows.at[pl.ds(j * CHUNK, CHUNK)], gsems[j]
        )
        for j in range(N_CHUNKS)
    ]
    for j in range(N_CHUNKS):
        gathers[j].wait()
        pltpu.async_copy(
            rows.at[pl.ds(j * CHUNK, CHUNK)],
            out_hbm.at[pl.ds(base + j * CHUNK, CHUNK)],
            wsem,
        )

    pltpu.make_async_copy(out_hbm.at[pl.ds(0, B_PER_W)], rows, wsem).wait()


@jax.jit
def kernel(t, time_emb):
    mesh = plsc.VectorSubcoreMesh(
        core_axis_name="c", subcore_axis_name="s", num_cores=NC, num_subcores=NS
    )
    run = pl.kernel(
        _emb_lookup_body,
        out_type=jax.ShapeDtypeStruct((BATCH, COND_DIM), jnp.float32),
        mesh=mesh,
        scratch_types=[
            pltpu.VMEM((B_PER_W,), jnp.int32),
            pltpu.VMEM((N_CHUNKS, CHUNK), jnp.int32),
            pltpu.VMEM((B_PER_W, COND_DIM), jnp.float32),
            pltpu.VMEM_SHARED((T_MAX, COND_DIM), jnp.float32),
            pltpu.SemaphoreType.DMA,
            pltpu.SemaphoreType.DMA,
            pltpu.SemaphoreType.DMA,
            pltpu.SemaphoreType.DMA,
            pltpu.SemaphoreType.DMA,
        ],
    )
    return run(t, time_emb)
